# baseline (speedup 1.0000x reference)
import functools

import jax
import jax.numpy as jnp
from jax import lax
from jax.experimental import pallas as pl
from jax.experimental.pallas import tpu as pltpu
from jax.experimental.pallas import tpu_sc as plsc

_ED = 64
_NB = 8
_NBLK = _NB * _NB * _NB
_BATCH = 1024
_NW = 32
_BPW = _NBLK // _NW
_GW = 128
_NCH = _BATCH // _GW
_NBUF = 4
_PAD = 129


def _sc_gather(table, gi_tiles):
    mesh = plsc.VectorSubcoreMesh(core_axis_name="c", subcore_axis_name="s")

    @functools.partial(
        pl.kernel,
        mesh=mesh,
        out_type=jax.ShapeDtypeStruct((4096, 8, 8, 128), jnp.float32),
        scratch_types=[
            pltpu.VMEM((_BPW, _NCH, _GW), jnp.int32),
            [pltpu.VMEM((_GW, _ED), jnp.float32) for _ in range(_NBUF)],
            [pltpu.VMEM((8, 8, _PAD), jnp.float32) for _ in range(2)],
            [pltpu.SemaphoreType.DMA for _ in range(_NBUF)],
            [pltpu.SemaphoreType.DMA for _ in range(2)],
            pltpu.SemaphoreType.DMA,
        ],
        compiler_params=pltpu.CompilerParams(
            use_tc_tiling_on_sc=False,
            needs_layout_passes=False,
            disable_bounds_checks=True,
        ),
    )
    def k(table_hbm, gi_hbm, out_hbm, idx_v, rows, stage, gsem, wsem, isem):
        wid = lax.axis_index("s") * 2 + lax.axis_index("c")
        pltpu.async_copy(gi_hbm.at[wid], idx_v, isem).wait()

        t = lax.iota(jnp.int32, 16)
        ehi = [((j * 16 + t) >> 3) for j in range(4)]
        elo = [((j * 16 + t) & 7) for j in range(4)]

        def gather(blk, g, rb):
            pltpu.async_copy(
                table_hbm.at[idx_v.at[blk, g]], rows[rb], gsem[rb])

        def gather_wait(blk, g, rb):
            pltpu.make_async_copy(
                table_hbm.at[idx_v.at[blk, g]], rows[rb], gsem[rb]).wait()

        def write_copy(blk, g, sb):
            base = pl.multiple_of((wid * _BPW + blk) * 8, 8)
            return pltpu.make_async_copy(
                stage[sb].at[:, :, pl.ds(0, 128)],
                out_hbm.at[pl.ds(base, 8), g],
                wsem[sb],
            )

        for p in range(_NBUF):
            gather(0, p, p)

        def block_body(blk, carry):
            def g2_body(g2, carry2):
                for gp in range(_NBUF):
                    g = g2 * _NBUF + gp
                    sb = gp % 2
                    gather_wait(blk, g, gp)

                    @pl.when(blk * _NCH + g >= 2)
                    def _():
                        write_copy(blk, g, sb).wait()

                    @plsc.parallel_loop(0, _GW, step=1, unroll=16)
                    def _(l):
                        lv = jnp.full((16,), l, jnp.int32)
                        for j in range(4):
                            v = rows[gp][l, pl.ds(j * 16, 16)]
                            plsc.store_scatter(
                                stage[sb], [ehi[j], elo[j], lv], v)

                    write_copy(blk, g, sb).start()

                    nc = blk * _NCH + g + _NBUF
                    nblk = nc // _NCH
                    ng = nc % _NCH

                    @pl.when(nblk < _BPW)
                    def _():
                        gather(nblk, ng, gp)
                return carry2

            lax.fori_loop(0, _NCH // _NBUF, g2_body, 0)
            return carry

        lax.fori_loop(0, _BPW, block_body, 0)
        for sb in range(2):
            write_copy(_BPW - 1, 6 + sb, sb).wait()

    return k(table, gi_tiles)


def kernel(x, gi, ee):
    table = jnp.concatenate([ee, x], axis=0)
    git = gi.astype(jnp.int32).T.reshape(_NW, _BPW, _NCH, _GW)
    buf = _sc_gather(table, git)
    r = buf.reshape(_NBLK, 8, 8, 8, 128)
    out = r.transpose(2, 4, 0, 1, 3)
    return out.reshape(gi.shape[0], _NB, _NB, _NB, _ED)
